# baseline (speedup 1.0000x reference)
import functools

import jax
import jax.numpy as jnp
from jax import lax
from jax.experimental import pallas as pl
from jax.experimental.pallas import tpu as pltpu
from jax.experimental.pallas import tpu_sc as plsc

NC = 2
NS = 16
NW = NC * NS


def _make_sc_edge_weights(E, N, EPP):
    EP = E // NW
    PAD = EPP - EP
    C1 = 2000
    NCHUNK = EP // C1
    NV = C1 // 16
    mesh = plsc.VectorSubcoreMesh(core_axis_name="c", subcore_axis_name="s")

    @functools.partial(
        pl.kernel,
        mesh=mesh,
        compiler_params=pltpu.CompilerParams(needs_layout_passes=False),
        out_type=[
            jax.ShapeDtypeStruct((NW * EPP,), jnp.float32),
            jax.ShapeDtypeStruct((NW * N,), jnp.float32),
        ],
        scratch_types=[
            pltpu.VMEM((16,), jnp.float32),
            [pltpu.VMEM((C1,), jnp.float32)] * NCHUNK,
            [pltpu.VMEM((C1,), jnp.int32)] * NCHUNK,
            [pltpu.VMEM((C1,), jnp.float32)] * NCHUNK,
            pltpu.VMEM((N,), jnp.float32),
            pltpu.VMEM((PAD,), jnp.float32),
            pltpu.SemaphoreType.DMA,
            pltpu.SemaphoreType.DMA,
        ],
    )
    def sc1(curva_hbm, col_hbm, wpack_hbm, ew_hbm, degp_hbm,
            wv, curv_b, col_b, ew_b, deg_l, zpad, fsem, wsem):
        c = lax.axis_index("c")
        s = lax.axis_index("s")
        wid = s * NC + c
        base = wid * EP
        obase = wid * EPP

        for k in range(NCHUNK):
            pltpu.async_copy(curva_hbm.at[pl.ds(base + k * C1, C1)],
                             curv_b[k], fsem)
            pltpu.async_copy(col_hbm.at[pl.ds(base + k * C1, C1)],
                             col_b[k], fsem)

        pltpu.sync_copy(wpack_hbm, wv)
        wsp = [plsc.load_gather(wv, [jnp.full((16,), k, jnp.int32)])
               for k in range(10)]
        csp = plsc.load_gather(wv, [jnp.full((16,), 10, jnp.int32)])

        zero16 = jnp.zeros((16,), jnp.float32)

        def zbody(i, _):
            deg_l[pl.ds(i * 16, 16)] = zero16
            return 0
        lax.fori_loop(0, N // 16, zbody, 0)
        for i in range(PAD // 16):
            zpad[pl.ds(i * 16, 16)] = zero16
        pltpu.async_copy(zpad, ew_hbm.at[pl.ds(obase + EP, PAD)], wsem)

        for k in range(NCHUNK):
            pltpu.make_async_copy(curva_hbm.at[pl.ds(base, C1)],
                                  curv_b[k], fsem).wait()
            pltpu.make_async_copy(col_hbm.at[pl.ds(base, C1)],
                                  col_b[k], fsem).wait()

        for k in range(NCHUNK):
            def vbody(i, _):
                cv = curv_b[k][pl.ds(i * 16, 16)]
                cols = col_b[k][pl.ds(i * 16, 16)]
                t = jnp.exp(-cv)
                p = wsp[9]
                for q in range(8, -1, -1):
                    p = p * t + wsp[q]
                ewv = csp + 0.5 * (t * p)
                ew_b[k][pl.ds(i * 16, 16)] = ewv
                plsc.addupdate_scatter(deg_l, [cols], ewv)
                return 0
            lax.fori_loop(0, NV, vbody, 0, unroll=2)
            pltpu.async_copy(ew_b[k], ew_hbm.at[pl.ds(obase + k * C1, C1)],
                             wsem)

        pltpu.sync_copy(deg_l, degp_hbm.at[pl.ds(wid * N, N)])
        pltpu.make_async_copy(zpad, ew_hbm.at[pl.ds(obase, PAD)],
                              wsem).wait()
        for k in range(NCHUNK):
            pltpu.make_async_copy(ew_b[k], ew_hbm.at[pl.ds(obase, C1)],
                                  wsem).wait()

    return sc1


def _tc_norm_linear(degp, x, W, ones_nw, NP):
    N, D = x.shape
    H = W.shape[1]

    def body(degp_ref, x_ref, w_ref, ones_ref, dinv_ref, xwn_ref):
        deg = lax.dot_general(degp_ref[...], ones_ref[...],
                              (((0,), (0,)), ((), ())),
                              preferred_element_type=jnp.float32) + 1.0
        safe = jnp.where(deg > 0.0, deg, 1.0)
        dinv = jnp.where(deg > 0.0, lax.rsqrt(safe), 0.0)
        xw = jnp.dot(x_ref[...], w_ref[...], preferred_element_type=jnp.float32)
        pad1 = jnp.zeros((NP - N, 1), jnp.float32)
        padh = jnp.zeros((NP - N, H), jnp.float32)
        dinv_ref[...] = jnp.concatenate([dinv, pad1], axis=0)
        xwn_ref[...] = jnp.concatenate([xw * dinv, padh], axis=0)

    return pl.pallas_call(
        body,
        out_shape=[
            jax.ShapeDtypeStruct((NP, 1), jnp.float32),
            jax.ShapeDtypeStruct((NP, H), jnp.float32),
        ],
    )(degp, x, W, ones_nw)


def _make_sc_scatter(EPP, NP, H):
    C2 = 128
    NCH = EPP // C2
    NPAIR = NCH // 2
    NPS = NP // NS
    NL = H // 16
    mesh = plsc.VectorSubcoreMesh(core_axis_name="c", subcore_axis_name="s")

    @functools.partial(
        pl.kernel,
        mesh=mesh,
        compiler_params=pltpu.CompilerParams(needs_layout_passes=False),
        out_type=jax.ShapeDtypeStruct((NC, NP, H), jnp.float32),
        scratch_types=[
            pltpu.VMEM_SHARED((NP, H), jnp.float32),
            pltpu.VMEM((EPP,), jnp.int32),
            pltpu.VMEM((C2, H), jnp.float32),
            pltpu.VMEM((C2, H), jnp.float32),
            pltpu.VMEM((C2,), jnp.int32),
            pltpu.VMEM((C2,), jnp.int32),
            pltpu.VMEM((C2,), jnp.float32),
            pltpu.VMEM((C2,), jnp.float32),
            pltpu.SemaphoreType.DMA,
            pltpu.SemaphoreType.DMA,
            pltpu.SemaphoreType.DMA,
            pltpu.SemaphoreType.DMA,
        ],
    )
    def sc2(xwn_hbm, row_hbm, col_hbm, ew_hbm, zeros_hbm, acc_hbm,
            acc_sh, rowb, r0, r1, cb0, cb1, eb0, eb1, g0, g1, s0, s1):
        c = lax.axis_index("c")
        s = lax.axis_index("s")
        wid = s * NC + c
        base = wid * EPP
        nsl = pl.ds(s * NPS, NPS)
        rows = (r0, r1)
        cbs = (cb0, cb1)
        ebs = (eb0, eb1)
        gs = (g0, g1)
        ss = (s0, s1)

        @pl.when(c == 0)
        def _():
            pltpu.sync_copy(xwn_hbm.at[nsl], acc_sh.at[nsl])

        @pl.when(c != 0)
        def _():
            pltpu.sync_copy(zeros_hbm.at[nsl], acc_sh.at[nsl])

        pltpu.sync_copy(row_hbm.at[pl.ds(base, EPP)], rowb)
        plsc.subcore_barrier()

        def start_fetch(k, j):
            pltpu.async_copy(col_hbm.at[pl.ds(base + k * C2, C2)],
                             cbs[j], gs[j])
            pltpu.async_copy(ew_hbm.at[pl.ds(base + k * C2, C2)],
                             ebs[j], gs[j])
            pltpu.async_copy(xwn_hbm.at[rowb.at[pl.ds(k * C2, C2)]],
                             rows[j], gs[j])

        def wait_fetch(j):
            pltpu.make_async_copy(col_hbm.at[pl.ds(base, C2)],
                                  cbs[j], gs[j]).wait()
            pltpu.make_async_copy(ew_hbm.at[pl.ds(base, C2)],
                                  ebs[j], gs[j]).wait()
            pltpu.make_async_copy(xwn_hbm.at[rowb.at[pl.ds(0, C2)]],
                                  rows[j], gs[j]).wait()

        def scale(j, k):
            @plsc.parallel_loop(0, C2, 1, unroll=4)
            def _(r):
                w = plsc.load_gather(
                    ebs[j], [jnp.full((16,), r, jnp.int32)])
                for q in range(NL):
                    rows[j][r, pl.ds(q * 16, 16)] = \
                        rows[j][r, pl.ds(q * 16, 16)] * w

        def start_scatter(j):
            pltpu.async_copy(rows[j], acc_sh.at[cbs[j]], ss[j], add=True)

        def wait_scatter(j):
            pltpu.make_async_copy(rows[j], acc_sh.at[cbs[j]], ss[j]).wait()

        start_fetch(0, 0)

        def pair(t, _):
            a = t * 2
            wait_fetch(0)

            @pl.when(t > 0)
            def _():
                wait_scatter(1)
            start_fetch(a + 1, 1)
            scale(0, a)
            start_scatter(0)
            wait_fetch(1)
            scale(1, a + 1)
            start_scatter(1)

            @pl.when(t < NPAIR - 1)
            def _():
                wait_scatter(0)
                start_fetch(a + 2, 0)
            return 0
        lax.fori_loop(0, NPAIR, pair, 0)
        wait_scatter(0)
        wait_scatter(1)

        plsc.subcore_barrier()
        pltpu.sync_copy(acc_sh.at[nsl], acc_hbm.at[c, nsl])

    return sc2


def _make_sc_gather(BT, NPR, H):
    RP = BT // NW
    C3 = 128
    NCHUNK = RP // C3
    NB = 4
    mesh = plsc.VectorSubcoreMesh(core_axis_name="c", subcore_axis_name="s")

    @functools.partial(
        pl.kernel,
        mesh=mesh,
        compiler_params=pltpu.CompilerParams(needs_layout_passes=False),
        out_type=[
            jax.ShapeDtypeStruct((BT, H), jnp.float32),
            jax.ShapeDtypeStruct((BT,), jnp.float32),
        ],
        scratch_types=[
            [pltpu.VMEM((C3,), jnp.int32)] * NCHUNK,
            pltpu.VMEM((NPR,), jnp.float32),
            pltpu.VMEM((BT // NW,), jnp.float32),
            pltpu.VMEM((C3, H), jnp.float32),
            pltpu.VMEM((C3, H), jnp.float32),
            pltpu.VMEM((C3, H), jnp.float32),
            pltpu.VMEM((C3, H), jnp.float32),
            pltpu.SemaphoreType.DMA,
            pltpu.SemaphoreType.DMA,
            pltpu.SemaphoreType.DMA,
            pltpu.SemaphoreType.DMA,
            pltpu.SemaphoreType.DMA,
            pltpu.SemaphoreType.DMA,
            pltpu.SemaphoreType.DMA,
            pltpu.SemaphoreType.DMA,
            pltpu.SemaphoreType.DMA,
            pltpu.SemaphoreType.DMA,
        ],
    )
    def sc3(acc0_hbm, acc1_hbm, dinv_hbm, idx_hbm, out_hbm, dout_hbm,
            ibs, dinv_v, dg, b0, b1, b2, b3,
            isem, dsem, g0, g1, g2, g3, w0, w1, w2, w3):
        c = lax.axis_index("c")
        s = lax.axis_index("s")
        wid = s * NC + c
        base = wid * RP
        bufs = (b0, b1, b2, b3)
        gs = (g0, g1, g2, g3)
        ws = (w0, w1, w2, w3)

        for k in range(NCHUNK):
            pltpu.async_copy(idx_hbm.at[pl.ds(base + k * C3, C3)],
                             ibs[k], isem)
        pltpu.async_copy(dinv_hbm, dinv_v, dsem)
        for k in range(NCHUNK):
            pltpu.make_async_copy(idx_hbm.at[pl.ds(base, C3)],
                                  ibs[k], isem).wait()
        pltpu.make_async_copy(dinv_hbm, dinv_v, dsem).wait()

        def start_g(k, j):
            pltpu.async_copy(acc0_hbm.at[ibs[k]], bufs[j], gs[j])
            pltpu.async_copy(acc1_hbm.at[ibs[k]], bufs[j], gs[j], add=True)

        def wait_g(j):
            pltpu.make_async_copy(acc0_hbm.at[ibs[0]], bufs[j], gs[j]).wait()
            pltpu.make_async_copy(acc1_hbm.at[ibs[0]], bufs[j], gs[j]).wait()

        def start_w(k, j):
            pltpu.async_copy(bufs[j], out_hbm.at[pl.ds(base + k * C3, C3)],
                             ws[j])

        def wait_w(j):
            pltpu.make_async_copy(bufs[j], out_hbm.at[pl.ds(base, C3)],
                                  ws[j]).wait()

        for k in range(NCHUNK):
            for g in range(C3 // 16):
                iv = ibs[k][pl.ds(g * 16, 16)]
                dg[pl.ds(k * C3 + g * 16, 16)] = \
                    plsc.load_gather(dinv_v, [iv])
        pltpu.sync_copy(dg, dout_hbm.at[pl.ds(base, RP)])

        for k in range(NB):
            start_g(k, k)
        for k in range(NCHUNK):
            j = k % NB
            wait_g(j)
            start_w(k, j)
            if k + NB < NCHUNK:
                wait_w(j)
                start_g(k + NB, j)
        for j in range(NB):
            wait_w(j)

    return sc3


def _tc_decoder(ec1, ec2, d1, d2, bg_2d, W_d1, b1_2d, W_d2, b2_2d):
    B, H = ec1.shape
    BLK = 2048
    grid = B // BLK

    def body(e1_ref, e2_ref, d1_ref, d2_ref, bg_ref,
             w1_ref, b1_ref, w2_ref, b2_ref, out_ref):
        a = jnp.maximum(e1_ref[...] * d1_ref[...] + bg_ref[...], 0.0)
        b = jnp.maximum(e2_ref[...] * d2_ref[...] + bg_ref[...], 0.0)
        feat = jnp.concatenate([a + b, a * b, a, b], axis=1)
        h = jnp.dot(feat, w1_ref[...], preferred_element_type=jnp.float32)
        h = jnp.maximum(h + b1_ref[...], 0.0)
        out_ref[...] = jnp.dot(h, w2_ref[...],
                               preferred_element_type=jnp.float32) + b2_ref[...]

    return pl.pallas_call(
        body,
        grid=(grid,),
        in_specs=[
            pl.BlockSpec((BLK, H), lambda i: (i, 0)),
            pl.BlockSpec((BLK, H), lambda i: (i, 0)),
            pl.BlockSpec((BLK, 1), lambda i: (i, 0)),
            pl.BlockSpec((BLK, 1), lambda i: (i, 0)),
            pl.BlockSpec(bg_2d.shape, lambda i: (0, 0)),
            pl.BlockSpec(W_d1.shape, lambda i: (0, 0)),
            pl.BlockSpec(b1_2d.shape, lambda i: (0, 0)),
            pl.BlockSpec(W_d2.shape, lambda i: (0, 0)),
            pl.BlockSpec(b2_2d.shape, lambda i: (0, 0)),
        ],
        out_specs=pl.BlockSpec((BLK, 1), lambda i: (i, 0)),
        out_shape=jax.ShapeDtypeStruct((B, 1), jnp.float32),
    )(ec1, ec2, d1, d2, bg_2d, W_d1, b1_2d, W_d2, b2_2d)


def kernel(x, edge_index, curva, idx, W_lin1, b_lin1, W_gcn, b_gcn,
           W_d1, b_d1, W_d2, b_d2):
    N, D = x.shape
    E = curva.shape[0]
    B = idx.shape[1]
    H = W_gcn.shape[1]
    NP = 10240

    EP = E // NW
    EPP = 10240
    row = edge_index[0]
    col = edge_index[1]
    pad2d = ((0, 0), (0, EPP - EP))
    row_p = jnp.pad(row.reshape(NW, EP), pad2d).reshape(NW * EPP)
    col_p = jnp.pad(col.reshape(NW, EP), pad2d).reshape(NW * EPP)

    c0 = b_lin1[0] + 0.5 * jnp.sum(W_lin1)
    wpack = jnp.concatenate(
        [W_lin1[:, 0], c0[None], jnp.zeros((5,), jnp.float32)])

    sc1 = _make_sc_edge_weights(E, N, EPP)
    ew, degp = sc1(curva, col, wpack)

    ones_nw = jnp.ones((NW, 1), jnp.float32)
    dinv, xwn = _tc_norm_linear(degp.reshape(NW, N), x, W_gcn, ones_nw, NP)

    sc2 = _make_sc_scatter(EPP, NP, H)
    accp = sc2(xwn, row_p, col_p, ew, jnp.zeros((NP, H), jnp.float32))

    sc3 = _make_sc_gather(2 * B, NP, H)
    e12, dgath = sc3(accp[0], accp[1], dinv.reshape(NP), idx.reshape(2 * B))

    log = _tc_decoder(e12[:B], e12[B:], dgath[:B, None], dgath[B:, None],
                      b_gcn[None, :], W_d1, b_d1[None, :], W_d2,
                      b_d2[None, :])
    return log


# speedup vs baseline: 2.1606x; 2.1606x over previous
import functools

import jax
import jax.numpy as jnp
from jax import lax
from jax.experimental import pallas as pl
from jax.experimental.pallas import tpu as pltpu
from jax.experimental.pallas import tpu_sc as plsc

NC = 2
NS = 16
NW = NC * NS


def _make_sc_edge_weights(E, N):
    EP = E // NW
    C1 = 2000
    NCHUNK = EP // C1
    NV = C1 // 16
    mesh = plsc.VectorSubcoreMesh(core_axis_name="c", subcore_axis_name="s")

    @functools.partial(
        pl.kernel,
        mesh=mesh,
        compiler_params=pltpu.CompilerParams(needs_layout_passes=False),
        out_type=[
            jax.ShapeDtypeStruct((E,), jnp.float32),
            jax.ShapeDtypeStruct((NW * N,), jnp.float32),
        ],
        scratch_types=[
            pltpu.VMEM((16,), jnp.float32),
            [pltpu.VMEM((C1,), jnp.float32)] * NCHUNK,
            [pltpu.VMEM((C1,), jnp.int32)] * NCHUNK,
            [pltpu.VMEM((C1,), jnp.float32)] * NCHUNK,
            pltpu.VMEM((N,), jnp.float32),
            pltpu.SemaphoreType.DMA,
            pltpu.SemaphoreType.DMA,
        ],
    )
    def sc1(curva_hbm, col_hbm, wpack_hbm, ew_hbm, degp_hbm,
            wv, curv_b, col_b, ew_b, deg_l, fsem, wsem):
        c = lax.axis_index("c")
        s = lax.axis_index("s")
        wid = s * NC + c
        base = wid * EP

        for k in range(NCHUNK):
            pltpu.async_copy(curva_hbm.at[pl.ds(base + k * C1, C1)],
                             curv_b[k], fsem)
            pltpu.async_copy(col_hbm.at[pl.ds(base + k * C1, C1)],
                             col_b[k], fsem)

        pltpu.sync_copy(wpack_hbm, wv)
        wsp = [plsc.load_gather(wv, [jnp.full((16,), k, jnp.int32)])
               for k in range(10)]
        csp = plsc.load_gather(wv, [jnp.full((16,), 10, jnp.int32)])

        zero16 = jnp.zeros((16,), jnp.float32)

        def zbody(i, _):
            deg_l[pl.ds(i * 16, 16)] = zero16
            return 0
        lax.fori_loop(0, N // 16, zbody, 0)

        for k in range(NCHUNK):
            pltpu.make_async_copy(curva_hbm.at[pl.ds(base, C1)],
                                  curv_b[k], fsem).wait()
            pltpu.make_async_copy(col_hbm.at[pl.ds(base, C1)],
                                  col_b[k], fsem).wait()

        for k in range(NCHUNK):
            def vbody(i, _):
                cv = curv_b[k][pl.ds(i * 16, 16)]
                cols = col_b[k][pl.ds(i * 16, 16)]
                t = jnp.exp(-cv)
                p = wsp[9]
                for q in range(8, -1, -1):
                    p = p * t + wsp[q]
                ewv = csp + 0.5 * (t * p)
                ew_b[k][pl.ds(i * 16, 16)] = ewv
                plsc.addupdate_scatter(deg_l, [cols], ewv)
                return 0
            lax.fori_loop(0, NV, vbody, 0, unroll=2)
            pltpu.async_copy(ew_b[k], ew_hbm.at[pl.ds(base + k * C1, C1)],
                             wsem)

        pltpu.sync_copy(deg_l, degp_hbm.at[pl.ds(wid * N, N)])
        for k in range(NCHUNK):
            pltpu.make_async_copy(ew_b[k], ew_hbm.at[pl.ds(base, C1)],
                                  wsem).wait()

    return sc1


def _tc_norm_linear(degp, x, W, ones_nw, NP):
    N, D = x.shape
    H = W.shape[1]

    def body(degp_ref, x_ref, w_ref, ones_ref, dinv_ref, xwn_ref):
        deg = lax.dot_general(degp_ref[...], ones_ref[...],
                              (((0,), (0,)), ((), ())),
                              preferred_element_type=jnp.float32) + 1.0
        safe = jnp.where(deg > 0.0, deg, 1.0)
        dinv = jnp.where(deg > 0.0, lax.rsqrt(safe), 0.0)
        xw = jnp.dot(x_ref[...], w_ref[...], preferred_element_type=jnp.float32)
        pad1 = jnp.zeros((NP - N, 1), jnp.float32)
        padh = jnp.zeros((NP - N, H), jnp.float32)
        dinv_ref[...] = jnp.concatenate([dinv, pad1], axis=0)
        xwn_ref[...] = jnp.concatenate([xw * dinv, padh], axis=0)

    return pl.pallas_call(
        body,
        out_shape=[
            jax.ShapeDtypeStruct((NP, 1), jnp.float32),
            jax.ShapeDtypeStruct((NP, H), jnp.float32),
        ],
    )(degp, x, W, ones_nw)


def _make_sc_scatter(E, NP, H):
    EP = E // NW
    C2 = 80
    NCH = EP // C2
    NT = (NCH - 2) // 3
    NPS = NP // NS
    NL = H // 16
    mesh = plsc.VectorSubcoreMesh(core_axis_name="c", subcore_axis_name="s")

    @functools.partial(
        pl.kernel,
        mesh=mesh,
        compiler_params=pltpu.CompilerParams(needs_layout_passes=False),
        out_type=jax.ShapeDtypeStruct((NC, NP, H), jnp.float32),
        scratch_types=[
            pltpu.VMEM_SHARED((NP, H), jnp.float32),
            pltpu.VMEM((EP,), jnp.int32),
            pltpu.VMEM((C2, H), jnp.float32),
            pltpu.VMEM((C2, H), jnp.float32),
            pltpu.VMEM((C2, H), jnp.float32),
            pltpu.VMEM((C2,), jnp.int32),
            pltpu.VMEM((C2,), jnp.int32),
            pltpu.VMEM((C2,), jnp.int32),
            pltpu.VMEM((C2,), jnp.float32),
            pltpu.VMEM((C2,), jnp.float32),
            pltpu.VMEM((C2,), jnp.float32),
            pltpu.SemaphoreType.DMA,
            pltpu.SemaphoreType.DMA,
            pltpu.SemaphoreType.DMA,
            pltpu.SemaphoreType.DMA,
            pltpu.SemaphoreType.DMA,
            pltpu.SemaphoreType.DMA,
        ],
    )
    def sc2(xwn_hbm, row_hbm, col_hbm, ew_hbm, zeros_hbm, acc_hbm,
            acc_sh, rowb, r0, r1, r2, cb0, cb1, cb2, eb0, eb1, eb2,
            g0, g1, g2, s0, s1, s2):
        c = lax.axis_index("c")
        s = lax.axis_index("s")
        wid = s * NC + c
        base = wid * EP
        nsl = pl.ds(s * NPS, NPS)
        rows = (r0, r1, r2)
        cbs = (cb0, cb1, cb2)
        ebs = (eb0, eb1, eb2)
        gs = (g0, g1, g2)
        ss = (s0, s1, s2)

        @pl.when(c == 0)
        def _():
            pltpu.sync_copy(xwn_hbm.at[nsl], acc_sh.at[nsl])

        @pl.when(c != 0)
        def _():
            pltpu.sync_copy(zeros_hbm.at[nsl], acc_sh.at[nsl])

        pltpu.sync_copy(row_hbm.at[pl.ds(base, EP)], rowb)
        plsc.subcore_barrier()

        def start_fetch(k, j):
            pltpu.async_copy(col_hbm.at[pl.ds(base + k * C2, C2)],
                             cbs[j], gs[j])
            pltpu.async_copy(ew_hbm.at[pl.ds(base + k * C2, C2)],
                             ebs[j], gs[j])
            pltpu.async_copy(xwn_hbm.at[rowb.at[pl.ds(k * C2, C2)]],
                             rows[j], gs[j])

        def wait_fetch(j):
            pltpu.make_async_copy(col_hbm.at[pl.ds(base, C2)],
                                  cbs[j], gs[j]).wait()
            pltpu.make_async_copy(ew_hbm.at[pl.ds(base, C2)],
                                  ebs[j], gs[j]).wait()
            pltpu.make_async_copy(xwn_hbm.at[rowb.at[pl.ds(0, C2)]],
                                  rows[j], gs[j]).wait()

        def scale(j, k):
            def rb(r, _):
                w = plsc.load_gather(
                    ebs[j], [jnp.full((16,), r, jnp.int32)])
                for q in range(NL):
                    rows[j][r, pl.ds(q * 16, 16)] = \
                        rows[j][r, pl.ds(q * 16, 16)] * w
                return 0
            lax.fori_loop(0, C2, rb, 0, unroll=4)

        def start_scatter(j):
            pltpu.async_copy(rows[j], acc_sh.at[cbs[j]], ss[j], add=True)

        def wait_scatter(j):
            pltpu.make_async_copy(rows[j], acc_sh.at[cbs[j]], ss[j]).wait()

        for j in range(3):
            start_fetch(j, j)

        def triple(t, _):
            for j in range(3):
                k = t * 3 + j
                wait_fetch(j)
                scale(j, k)
                start_scatter(j)
                jp = (j - 1) % 3
                if j == 0:
                    @pl.when(t > 0)
                    def _():
                        wait_scatter(jp)
                        start_fetch(k + 2, jp)
                else:
                    wait_scatter(jp)
                    start_fetch(k + 2, jp)
            return 0
        lax.fori_loop(0, NT, triple, 0)

        for j, k in ((0, NT * 3), (1, NT * 3 + 1)):
            wait_fetch(j)
            scale(j, k)
            start_scatter(j)
        wait_scatter(2)
        wait_scatter(0)
        wait_scatter(1)

        plsc.subcore_barrier()
        pltpu.sync_copy(acc_sh.at[nsl], acc_hbm.at[c, nsl])

    return sc2


def _tc_combine_relu(accp, dinv, b2d):
    _, NP, H = accp.shape

    def body(acc_ref, dinv_ref, b_ref, out_ref):
        tot = acc_ref[0] + acc_ref[1]
        out_ref[...] = jnp.maximum(tot * dinv_ref[...] + b_ref[...], 0.0)

    return pl.pallas_call(
        body,
        out_shape=jax.ShapeDtypeStruct((NP, H), jnp.float32),
    )(accp, dinv, b2d)


def _make_sc_gather(BT, NPR, H):
    RP = BT // NW
    C3 = 128
    NCHUNK = RP // C3
    NB = 4
    mesh = plsc.VectorSubcoreMesh(core_axis_name="c", subcore_axis_name="s")

    @functools.partial(
        pl.kernel,
        mesh=mesh,
        compiler_params=pltpu.CompilerParams(needs_layout_passes=False),
        out_type=jax.ShapeDtypeStruct((BT, H), jnp.float32),
        scratch_types=[
            [pltpu.VMEM((C3,), jnp.int32)] * NCHUNK,
            pltpu.VMEM((C3, H), jnp.float32),
            pltpu.VMEM((C3, H), jnp.float32),
            pltpu.VMEM((C3, H), jnp.float32),
            pltpu.VMEM((C3, H), jnp.float32),
            pltpu.SemaphoreType.DMA,
            pltpu.SemaphoreType.DMA,
            pltpu.SemaphoreType.DMA,
            pltpu.SemaphoreType.DMA,
            pltpu.SemaphoreType.DMA,
            pltpu.SemaphoreType.DMA,
            pltpu.SemaphoreType.DMA,
            pltpu.SemaphoreType.DMA,
            pltpu.SemaphoreType.DMA,
        ],
    )
    def sc3(x1_hbm, idx_hbm, out_hbm, ibs, b0, b1, b2, b3,
            isem, g0, g1, g2, g3, w0, w1, w2, w3):
        c = lax.axis_index("c")
        s = lax.axis_index("s")
        wid = s * NC + c
        base = wid * RP
        bufs = (b0, b1, b2, b3)
        gs = (g0, g1, g2, g3)
        ws = (w0, w1, w2, w3)

        for k in range(NCHUNK):
            pltpu.async_copy(idx_hbm.at[pl.ds(base + k * C3, C3)],
                             ibs[k], isem)
        for k in range(NCHUNK):
            pltpu.make_async_copy(idx_hbm.at[pl.ds(base, C3)],
                                  ibs[k], isem).wait()

        def start_g(k, j):
            pltpu.async_copy(x1_hbm.at[ibs[k]], bufs[j], gs[j])

        def wait_g(j):
            pltpu.make_async_copy(x1_hbm.at[ibs[0]], bufs[j], gs[j]).wait()

        def start_w(k, j):
            pltpu.async_copy(bufs[j], out_hbm.at[pl.ds(base + k * C3, C3)],
                             ws[j])

        def wait_w(j):
            pltpu.make_async_copy(bufs[j], out_hbm.at[pl.ds(base, C3)],
                                  ws[j]).wait()

        for k in range(NB):
            start_g(k, k)
        for k in range(NCHUNK):
            j = k % NB
            wait_g(j)
            start_w(k, j)
            if k + NB < NCHUNK:
                wait_w(j)
                start_g(k + NB, j)
        for j in range(NB):
            wait_w(j)

    return sc3


def _tc_decoder(e1, e2, W_d1, b1_2d, W_d2, b2_2d):
    B, H = e1.shape
    BLK = 2048
    grid = B // BLK

    def body(e1_ref, e2_ref, w1_ref, b1_ref, w2_ref, b2_ref, out_ref):
        a = e1_ref[...]
        b = e2_ref[...]
        feat = jnp.concatenate([a + b, a * b, a, b], axis=1)
        h = jnp.dot(feat, w1_ref[...], preferred_element_type=jnp.float32)
        h = jnp.maximum(h + b1_ref[...], 0.0)
        out_ref[...] = jnp.dot(h, w2_ref[...],
                               preferred_element_type=jnp.float32) + b2_ref[...]

    return pl.pallas_call(
        body,
        grid=(grid,),
        in_specs=[
            pl.BlockSpec((BLK, H), lambda i: (i, 0)),
            pl.BlockSpec((BLK, H), lambda i: (i, 0)),
            pl.BlockSpec(W_d1.shape, lambda i: (0, 0)),
            pl.BlockSpec(b1_2d.shape, lambda i: (0, 0)),
            pl.BlockSpec(W_d2.shape, lambda i: (0, 0)),
            pl.BlockSpec(b2_2d.shape, lambda i: (0, 0)),
        ],
        out_specs=pl.BlockSpec((BLK, 1), lambda i: (i, 0)),
        out_shape=jax.ShapeDtypeStruct((B, 1), jnp.float32),
    )(e1, e2, W_d1, b1_2d, W_d2, b2_2d)


def kernel(x, edge_index, curva, idx, W_lin1, b_lin1, W_gcn, b_gcn,
           W_d1, b_d1, W_d2, b_d2):
    N, D = x.shape
    E = curva.shape[0]
    B = idx.shape[1]
    H = W_gcn.shape[1]
    NP = 10240

    row = edge_index[0]
    col = edge_index[1]

    c0 = b_lin1[0] + 0.5 * jnp.sum(W_lin1)
    wpack = jnp.concatenate(
        [W_lin1[:, 0], c0[None], jnp.zeros((5,), jnp.float32)])

    sc1 = _make_sc_edge_weights(E, N)
    ew, degp = sc1(curva, col, wpack)

    ones_nw = jnp.ones((NW, 1), jnp.float32)
    dinv, xwn = _tc_norm_linear(degp.reshape(NW, N), x, W_gcn, ones_nw, NP)

    sc2 = _make_sc_scatter(E, NP, H)
    accp = sc2(xwn, row, col, ew, jnp.zeros((NP, H), jnp.float32))

    x1 = _tc_combine_relu(accp, dinv, b_gcn[None, :])

    sc3 = _make_sc_gather(2 * B, NP, H)
    e12 = sc3(x1, idx.reshape(2 * B))

    log = _tc_decoder(e12[:B], e12[B:], W_d1, b_d1[None, :], W_d2,
                      b_d2[None, :])
    return log


# speedup vs baseline: 2.2516x; 1.0421x over previous
import functools

import jax
import jax.numpy as jnp
from jax import lax
from jax.experimental import pallas as pl
from jax.experimental.pallas import tpu as pltpu
from jax.experimental.pallas import tpu_sc as plsc

NC = 2
NS = 16
NW = NC * NS


def _make_sc_edge_weights(E, N):
    EP = E // NW
    C1 = 2000
    NCHUNK = EP // C1
    NV = C1 // 16
    mesh = plsc.VectorSubcoreMesh(core_axis_name="c", subcore_axis_name="s")

    @functools.partial(
        pl.kernel,
        mesh=mesh,
        compiler_params=pltpu.CompilerParams(needs_layout_passes=False),
        out_type=[
            jax.ShapeDtypeStruct((E,), jnp.float32),
            jax.ShapeDtypeStruct((NW * N,), jnp.float32),
        ],
        scratch_types=[
            pltpu.VMEM((16,), jnp.float32),
            [pltpu.VMEM((C1,), jnp.float32)] * NCHUNK,
            [pltpu.VMEM((C1,), jnp.int32)] * NCHUNK,
            [pltpu.VMEM((C1,), jnp.float32)] * NCHUNK,
            pltpu.VMEM((N,), jnp.float32),
            pltpu.SemaphoreType.DMA,
            pltpu.SemaphoreType.DMA,
        ],
    )
    def sc1(curva_hbm, col_hbm, wpack_hbm, ew_hbm, degp_hbm,
            wv, curv_b, col_b, ew_b, deg_l, fsem, wsem):
        c = lax.axis_index("c")
        s = lax.axis_index("s")
        wid = s * NC + c
        base = wid * EP

        for k in range(NCHUNK):
            pltpu.async_copy(curva_hbm.at[pl.ds(base + k * C1, C1)],
                             curv_b[k], fsem)
            pltpu.async_copy(col_hbm.at[pl.ds(base + k * C1, C1)],
                             col_b[k], fsem)

        pltpu.sync_copy(wpack_hbm, wv)
        wsp = [plsc.load_gather(wv, [jnp.full((16,), k, jnp.int32)])
               for k in range(10)]
        csp = plsc.load_gather(wv, [jnp.full((16,), 10, jnp.int32)])

        zero16 = jnp.zeros((16,), jnp.float32)

        def zbody(i, _):
            deg_l[pl.ds(i * 16, 16)] = zero16
            return 0
        lax.fori_loop(0, N // 16, zbody, 0)

        for k in range(NCHUNK):
            pltpu.make_async_copy(curva_hbm.at[pl.ds(base, C1)],
                                  curv_b[k], fsem).wait()
            pltpu.make_async_copy(col_hbm.at[pl.ds(base, C1)],
                                  col_b[k], fsem).wait()

        for k in range(NCHUNK):
            def vbody(i, _):
                cv = curv_b[k][pl.ds(i * 16, 16)]
                cols = col_b[k][pl.ds(i * 16, 16)]
                t = jnp.exp(-cv)
                p = wsp[9]
                for q in range(8, -1, -1):
                    p = p * t + wsp[q]
                ewv = csp + 0.5 * (t * p)
                ew_b[k][pl.ds(i * 16, 16)] = ewv
                plsc.addupdate_scatter(deg_l, [cols], ewv)
                return 0
            lax.fori_loop(0, NV, vbody, 0, unroll=2)
            pltpu.async_copy(ew_b[k], ew_hbm.at[pl.ds(base + k * C1, C1)],
                             wsem)

        pltpu.sync_copy(deg_l, degp_hbm.at[pl.ds(wid * N, N)])
        for k in range(NCHUNK):
            pltpu.make_async_copy(ew_b[k], ew_hbm.at[pl.ds(base, C1)],
                                  wsem).wait()

    return sc1


def _tc_norm_linear(degp, x, W, ones_nw, NP):
    N, D = x.shape
    H = W.shape[1]

    def body(degp_ref, x_ref, w_ref, ones_ref, dinv_ref, xwn_ref):
        deg = lax.dot_general(degp_ref[...], ones_ref[...],
                              (((0,), (0,)), ((), ())),
                              preferred_element_type=jnp.float32) + 1.0
        safe = jnp.where(deg > 0.0, deg, 1.0)
        dinv = jnp.where(deg > 0.0, lax.rsqrt(safe), 0.0)
        xw = jnp.dot(x_ref[...], w_ref[...], preferred_element_type=jnp.float32)
        pad1 = jnp.zeros((NP - N, 1), jnp.float32)
        padh = jnp.zeros((NP - N, H), jnp.float32)
        dinv_ref[...] = jnp.concatenate([dinv, pad1], axis=0)
        xwn_ref[...] = jnp.concatenate([xw * dinv, padh], axis=0)

    return pl.pallas_call(
        body,
        out_shape=[
            jax.ShapeDtypeStruct((NP, 1), jnp.float32),
            jax.ShapeDtypeStruct((NP, H), jnp.float32),
        ],
    )(degp, x, W, ones_nw)


def _make_sc_scatter(E, NP, H):
    EP = E // NW
    C2 = 80
    NCH = EP // C2
    NT = (NCH - 2) // 3
    NPS = NP // NS
    NL = H // 16
    mesh = plsc.VectorSubcoreMesh(core_axis_name="c", subcore_axis_name="s")

    @functools.partial(
        pl.kernel,
        mesh=mesh,
        compiler_params=pltpu.CompilerParams(needs_layout_passes=False),
        out_type=jax.ShapeDtypeStruct((NC, NP, H), jnp.float32),
        scratch_types=[
            pltpu.VMEM_SHARED((NP, H), jnp.float32),
            pltpu.VMEM((EP,), jnp.int32),
            pltpu.VMEM((C2, H), jnp.float32),
            pltpu.VMEM((C2, H), jnp.float32),
            pltpu.VMEM((C2, H), jnp.float32),
            pltpu.VMEM((C2,), jnp.int32),
            pltpu.VMEM((C2,), jnp.int32),
            pltpu.VMEM((C2,), jnp.int32),
            pltpu.VMEM((C2,), jnp.float32),
            pltpu.VMEM((C2,), jnp.float32),
            pltpu.VMEM((C2,), jnp.float32),
            pltpu.SemaphoreType.DMA,
            pltpu.SemaphoreType.DMA,
            pltpu.SemaphoreType.DMA,
            pltpu.SemaphoreType.DMA,
            pltpu.SemaphoreType.DMA,
            pltpu.SemaphoreType.DMA,
        ],
    )
    def sc2(xwn_hbm, row_hbm, col_hbm, ew_hbm, zeros_hbm, acc_hbm,
            acc_sh, rowb, r0, r1, r2, cb0, cb1, cb2, eb0, eb1, eb2,
            g0, g1, g2, s0, s1, s2):
        c = lax.axis_index("c")
        s = lax.axis_index("s")
        wid = s * NC + c
        base = wid * EP
        nsl = pl.ds(s * NPS, NPS)
        rows = (r0, r1, r2)
        cbs = (cb0, cb1, cb2)
        ebs = (eb0, eb1, eb2)
        gs = (g0, g1, g2)
        ss = (s0, s1, s2)

        @pl.when(c == 0)
        def _():
            pltpu.sync_copy(xwn_hbm.at[nsl], acc_sh.at[nsl])

        @pl.when(c != 0)
        def _():
            pltpu.sync_copy(zeros_hbm.at[nsl], acc_sh.at[nsl])

        pltpu.sync_copy(row_hbm.at[pl.ds(base, EP)], rowb)
        plsc.subcore_barrier()

        def start_fetch(k, j):
            pltpu.async_copy(col_hbm.at[pl.ds(base + k * C2, C2)],
                             cbs[j], gs[j])
            pltpu.async_copy(ew_hbm.at[pl.ds(base + k * C2, C2)],
                             ebs[j], gs[j])
            pltpu.async_copy(xwn_hbm.at[rowb.at[pl.ds(k * C2, C2)]],
                             rows[j], gs[j])

        def wait_fetch(j):
            pltpu.make_async_copy(col_hbm.at[pl.ds(base, C2)],
                                  cbs[j], gs[j]).wait()
            pltpu.make_async_copy(ew_hbm.at[pl.ds(base, C2)],
                                  ebs[j], gs[j]).wait()
            pltpu.make_async_copy(xwn_hbm.at[rowb.at[pl.ds(0, C2)]],
                                  rows[j], gs[j]).wait()

        def scale(j, k):
            def rb(r, _):
                w = plsc.load_gather(
                    ebs[j], [jnp.full((16,), r, jnp.int32)])
                for q in range(NL):
                    rows[j][r, pl.ds(q * 16, 16)] = \
                        rows[j][r, pl.ds(q * 16, 16)] * w
                return 0
            lax.fori_loop(0, C2, rb, 0, unroll=4)

        def start_scatter(j):
            pltpu.async_copy(rows[j], acc_sh.at[cbs[j]], ss[j], add=True)

        def wait_scatter(j):
            pltpu.make_async_copy(rows[j], acc_sh.at[cbs[j]], ss[j]).wait()

        for j in range(3):
            start_fetch(j, j)

        def triple(t, _):
            for j in range(3):
                k = t * 3 + j
                wait_fetch(j)
                scale(j, k)
                start_scatter(j)
                jp = (j - 1) % 3
                if j == 0:
                    @pl.when(t > 0)
                    def _():
                        wait_scatter(jp)
                        start_fetch(k + 2, jp)
                else:
                    wait_scatter(jp)
                    start_fetch(k + 2, jp)
            return 0
        lax.fori_loop(0, NT, triple, 0)

        for j, k in ((0, NT * 3), (1, NT * 3 + 1)):
            wait_fetch(j)
            scale(j, k)
            start_scatter(j)
        wait_scatter(2)
        wait_scatter(0)
        wait_scatter(1)

        plsc.subcore_barrier()
        pltpu.sync_copy(acc_sh.at[nsl], acc_hbm.at[c, nsl])

    return sc2


def _tc_combine_relu(accp, dinv, b2d):
    _, NP, H = accp.shape

    def body(acc_ref, dinv_ref, b_ref, out_ref):
        tot = acc_ref[0] + acc_ref[1]
        out_ref[...] = jnp.maximum(tot * dinv_ref[...] + b_ref[...], 0.0)

    return pl.pallas_call(
        body,
        out_shape=jax.ShapeDtypeStruct((NP, H), jnp.float32),
    )(accp, dinv, b2d)


def _make_sc_gather(BT, NPR, H):
    RP = BT // NW
    C3 = 128
    NCHUNK = RP // C3
    NB = 4
    mesh = plsc.VectorSubcoreMesh(core_axis_name="c", subcore_axis_name="s")

    @functools.partial(
        pl.kernel,
        mesh=mesh,
        compiler_params=pltpu.CompilerParams(needs_layout_passes=False),
        out_type=jax.ShapeDtypeStruct((BT, H), jnp.float32),
        scratch_types=[
            [pltpu.VMEM((C3,), jnp.int32)] * NCHUNK,
            pltpu.VMEM((C3, H), jnp.float32),
            pltpu.VMEM((C3, H), jnp.float32),
            pltpu.VMEM((C3, H), jnp.float32),
            pltpu.VMEM((C3, H), jnp.float32),
            pltpu.SemaphoreType.DMA,
            pltpu.SemaphoreType.DMA,
            pltpu.SemaphoreType.DMA,
            pltpu.SemaphoreType.DMA,
            pltpu.SemaphoreType.DMA,
            pltpu.SemaphoreType.DMA,
            pltpu.SemaphoreType.DMA,
            pltpu.SemaphoreType.DMA,
            pltpu.SemaphoreType.DMA,
        ],
    )
    def sc3(x1_hbm, idx_hbm, out_hbm, ibs, b0, b1, b2, b3,
            isem, g0, g1, g2, g3, w0, w1, w2, w3):
        c = lax.axis_index("c")
        s = lax.axis_index("s")
        wid = s * NC + c
        base = wid * RP
        bufs = (b0, b1, b2, b3)
        gs = (g0, g1, g2, g3)
        ws = (w0, w1, w2, w3)

        for k in range(NCHUNK):
            pltpu.async_copy(idx_hbm.at[pl.ds(base + k * C3, C3)],
                             ibs[k], isem)
        for k in range(NCHUNK):
            pltpu.make_async_copy(idx_hbm.at[pl.ds(base, C3)],
                                  ibs[k], isem).wait()

        def start_g(k, j):
            pltpu.async_copy(x1_hbm.at[ibs[k]], bufs[j], gs[j])

        def wait_g(j):
            pltpu.make_async_copy(x1_hbm.at[ibs[0]], bufs[j], gs[j]).wait()

        def start_w(k, j):
            pltpu.async_copy(bufs[j], out_hbm.at[pl.ds(base + k * C3, C3)],
                             ws[j])

        def wait_w(j):
            pltpu.make_async_copy(bufs[j], out_hbm.at[pl.ds(base, C3)],
                                  ws[j]).wait()

        for k in range(NB):
            start_g(k, k)
        for k in range(NCHUNK):
            j = k % NB
            wait_g(j)
            start_w(k, j)
            if k + NB < NCHUNK:
                wait_w(j)
                start_g(k + NB, j)
        for j in range(NB):
            wait_w(j)

    return sc3


def _tc_decoder(e12, B, W_d1, b1_2d, W_d2, b2_2d):
    H = e12.shape[1]
    BLK = 2048
    grid = B // BLK
    goff = B // BLK

    def body(e1_ref, e2_ref, w1_ref, b1_ref, w2_ref, b2_ref, out_ref):
        a = e1_ref[...]
        b = e2_ref[...]
        feat = jnp.concatenate([a + b, a * b, a, b], axis=1)
        h = jnp.dot(feat, w1_ref[...], preferred_element_type=jnp.float32)
        h = jnp.maximum(h + b1_ref[...], 0.0)
        out_ref[...] = jnp.dot(h, w2_ref[...],
                               preferred_element_type=jnp.float32) + b2_ref[...]

    return pl.pallas_call(
        body,
        grid=(grid,),
        in_specs=[
            pl.BlockSpec((BLK, H), lambda i: (i, 0)),
            pl.BlockSpec((BLK, H), lambda i: (i + goff, 0)),
            pl.BlockSpec(W_d1.shape, lambda i: (0, 0)),
            pl.BlockSpec(b1_2d.shape, lambda i: (0, 0)),
            pl.BlockSpec(W_d2.shape, lambda i: (0, 0)),
            pl.BlockSpec(b2_2d.shape, lambda i: (0, 0)),
        ],
        out_specs=pl.BlockSpec((BLK, 1), lambda i: (i, 0)),
        out_shape=jax.ShapeDtypeStruct((B, 1), jnp.float32),
    )(e12, e12, W_d1, b1_2d, W_d2, b2_2d)


def kernel(x, edge_index, curva, idx, W_lin1, b_lin1, W_gcn, b_gcn,
           W_d1, b_d1, W_d2, b_d2):
    N, D = x.shape
    E = curva.shape[0]
    B = idx.shape[1]
    H = W_gcn.shape[1]
    NP = 10240

    row = edge_index[0]
    col = edge_index[1]

    c0 = b_lin1[0] + 0.5 * jnp.sum(W_lin1)
    wpack = jnp.concatenate(
        [W_lin1[:, 0], c0[None], jnp.zeros((5,), jnp.float32)])

    sc1 = _make_sc_edge_weights(E, N)
    ew, degp = sc1(curva, col, wpack)

    ones_nw = jnp.ones((NW, 1), jnp.float32)
    dinv, xwn = _tc_norm_linear(degp.reshape(NW, N), x, W_gcn, ones_nw, NP)

    sc2 = _make_sc_scatter(E, NP, H)
    accp = sc2(xwn, row, col, ew, jnp.zeros((NP, H), jnp.float32))

    x1 = _tc_combine_relu(accp, dinv, b_gcn[None, :])

    sc3 = _make_sc_gather(2 * B, NP, H)
    e12 = sc3(x1, idx.reshape(2 * B))

    log = _tc_decoder(e12, B, W_d1, b_d1[None, :], W_d2, b_d2[None, :])
    return log
